# Initial kernel scaffold; baseline (speedup 1.0000x reference)
import jax
import jax.numpy as jnp
from jax.experimental import pallas as pl


def kernel(x, pos_table):
    raise NotImplementedError("write your pallas kernel here")


# speedup vs baseline: 1.4945x; 1.4945x over previous
import jax
import jax.numpy as jnp
from jax.experimental import pallas as pl


def _add_body(x_ref, pos_ref, o_ref):
    o_ref[...] = x_ref[...] + pos_ref[...]


def kernel(x, pos_table):
    batch, maxlen, embed = x.shape
    t_chunk = 512
    grid = (maxlen // t_chunk, batch)
    return pl.pallas_call(
        _add_body,
        grid=grid,
        in_specs=[
            pl.BlockSpec((1, t_chunk, embed), lambda t, b: (b, t, 0)),
            pl.BlockSpec((t_chunk, embed), lambda t, b: (t, 0)),
        ],
        out_specs=pl.BlockSpec((1, t_chunk, embed), lambda t, b: (b, t, 0)),
        out_shape=jax.ShapeDtypeStruct((batch, maxlen, embed), x.dtype),
    )(x, pos_table)
